# baseline (speedup 1.0000x reference)
import functools

import jax
import jax.numpy as jnp
from jax import lax
from jax.experimental import pallas as pl
from jax.experimental.pallas import tpu as pltpu
from jax.experimental.pallas import tpu_sc as plsc

NEG = 0.2
BN_EPS = 1e-5
NC = 2
NS = 16
LANES = 16

_HI = lax.Precision.HIGHEST


def _tc_lin(x, Wl, Wr):
    n, din = x.shape
    dout = Wl.shape[1]
    bn = 1000
    grid = (n // bn,)

    def body(x_ref, wl_ref, wr_ref, xl_ref, xr_ref):
        xb = x_ref[...]
        xl_ref[...] = lax.dot(xb, wl_ref[...], precision=_HI,
                              preferred_element_type=jnp.float32)
        xr_ref[...] = lax.dot(xb, wr_ref[...], precision=_HI,
                              preferred_element_type=jnp.float32)

    return pl.pallas_call(
        body,
        grid=grid,
        in_specs=[
            pl.BlockSpec((bn, din), lambda i: (i, 0)),
            pl.BlockSpec((din, dout), lambda i: (0, 0)),
            pl.BlockSpec((din, dout), lambda i: (0, 0)),
        ],
        out_specs=[
            pl.BlockSpec((bn, dout), lambda i: (i, 0)),
            pl.BlockSpec((bn, dout), lambda i: (i, 0)),
        ],
        out_shape=[
            jax.ShapeDtypeStruct((n, dout), jnp.float32),
            jax.ShapeDtypeStruct((n, dout), jnp.float32),
        ],
    )(x, Wl, Wr)


def _tc_combine1(xl1, xr1, num0, num1, den0, den1, A1, E8, scale, shift,
                 Wl2, Wr2):
    n = xl1.shape[0]
    bn = 1000
    grid = (n // bn,)

    def body(xl_ref, xr_ref, n0_ref, n1_ref, d0_ref, d1_ref, a1_ref, e8_ref,
             sc_ref, sh_ref, wl2_ref, wr2_ref, xl2_ref, xr2_ref):
        xl = xl_ref[...]
        t = xl + xr_ref[...]
        t = jnp.maximum(t, NEG * t)
        p = jnp.exp(lax.dot(t, a1_ref[...], precision=_HI,
                            preferred_element_type=jnp.float32))
        den8 = d0_ref[...] + d1_ref[...] + p
        e8 = e8_ref[...]
        num = n0_ref[...] + n1_ref[...] + lax.dot(
            p, e8, precision=_HI, preferred_element_type=jnp.float32) * xl
        den = lax.dot(den8, e8, precision=_HI,
                      preferred_element_type=jnp.float32)
        o = num / (den + 1e-16)
        o = o * sc_ref[...] + sh_ref[...]
        h = jnp.where(o > 0, o, jnp.exp(o) - 1.0)
        xl2_ref[...] = lax.dot(h, wl2_ref[...], precision=_HI,
                               preferred_element_type=jnp.float32)
        xr2_ref[...] = lax.dot(h, wr2_ref[...], precision=_HI,
                               preferred_element_type=jnp.float32)

    full = lambda shape: pl.BlockSpec(shape, lambda i: tuple(0 for _ in shape))
    return pl.pallas_call(
        body,
        grid=grid,
        in_specs=[
            pl.BlockSpec((bn, 128), lambda i: (i, 0)),
            pl.BlockSpec((bn, 128), lambda i: (i, 0)),
            pl.BlockSpec((bn, 128), lambda i: (i, 0)),
            pl.BlockSpec((bn, 128), lambda i: (i, 0)),
            pl.BlockSpec((bn, 8), lambda i: (i, 0)),
            pl.BlockSpec((bn, 8), lambda i: (i, 0)),
            full((128, 8)),
            full((8, 128)),
            full((1, 128)),
            full((1, 128)),
            full((128, 64)),
            full((128, 64)),
        ],
        out_specs=[
            pl.BlockSpec((bn, 64), lambda i: (i, 0)),
            pl.BlockSpec((bn, 64), lambda i: (i, 0)),
        ],
        out_shape=[
            jax.ShapeDtypeStruct((n, 64), jnp.float32),
            jax.ShapeDtypeStruct((n, 64), jnp.float32),
        ],
    )(xl1, xr1, num0, num1, den0, den1, A1, E8, scale, shift, Wl2, Wr2)


def _tc_final(xl2, xr2, num0, num1, den0, den1, att2, bias2):
    n = xl2.shape[0]
    bn = 1000
    grid = (n // bn,)

    def body(xl_ref, xr_ref, n0_ref, n1_ref, d0_ref, d1_ref, a_ref, b_ref,
             out_ref):
        xl = xl_ref[...]
        t = xl + xr_ref[...]
        t = jnp.maximum(t, NEG * t)
        a = jnp.sum(t * a_ref[...], axis=1, keepdims=True)
        p = jnp.exp(a)
        den = d0_ref[...] + d1_ref[...] + p
        num = n0_ref[...] + n1_ref[...] + p * xl
        o = num / (den + 1e-16) + b_ref[...]
        m = jnp.max(o, axis=1, keepdims=True)
        lse = m + jnp.log(jnp.sum(jnp.exp(o - m), axis=1, keepdims=True))
        out_ref[...] = o - lse

    full = lambda shape: pl.BlockSpec(shape, lambda i: tuple(0 for _ in shape))
    return pl.pallas_call(
        body,
        grid=grid,
        in_specs=[
            pl.BlockSpec((bn, 64), lambda i: (i, 0)),
            pl.BlockSpec((bn, 64), lambda i: (i, 0)),
            pl.BlockSpec((bn, 64), lambda i: (i, 0)),
            pl.BlockSpec((bn, 64), lambda i: (i, 0)),
            pl.BlockSpec((bn, 1), lambda i: (i, 0)),
            pl.BlockSpec((bn, 1), lambda i: (i, 0)),
            full((1, 64)),
            full((1, 64)),
        ],
        out_specs=pl.BlockSpec((bn, 64), lambda i: (i, 0)),
        out_shape=jax.ShapeDtypeStruct((n, 64), jnp.float32),
    )(xl2, xr2, num0, num1, den0, den1, att2, bias2)


_GDN = lax.GatherDimensionNumbers(
    offset_dims=(), collapsed_slice_dims=(0,), start_index_map=(0,))


def _lane_shuffle(u, idx):
    return lax.gather(u, idx[:, None], _GDN, (1,),
                      mode=lax.GatherScatterMode.PROMISE_IN_BOUNDS)


def _vsum_bcast(u, iota):
    for k in (1, 2, 4, 8):
        u = u + _lane_shuffle(u, iota ^ k)
    return u

def _sc_edges(xl, xr, src, dst, att, heads, chunk, single_head):
    n = xl.shape[0]
    dim = heads * LANES
    acc_w = dim + LANES
    e = src.shape[0]
    ew = e // (NC * NS)
    nchunks = ew // chunk
    assert nchunks % 2 == 0
    rows = n // NS

    mesh = plsc.VectorSubcoreMesh(core_axis_name="c", subcore_axis_name="s")

    @functools.partial(
        pl.kernel,
        out_type=jax.ShapeDtypeStruct((NC, n, acc_w), jnp.float32),
        mesh=mesh,
        compiler_params=pltpu.CompilerParams(use_tc_tiling_on_sc=False),
        scratch_types=[
            pltpu.VMEM((chunk,), jnp.int32),
            pltpu.VMEM((chunk,), jnp.int32),
            pltpu.VMEM((chunk,), jnp.int32),
            pltpu.VMEM((chunk,), jnp.int32),
            pltpu.VMEM((chunk, dim), jnp.float32),
            pltpu.VMEM((chunk, dim), jnp.float32),
            pltpu.VMEM((chunk, dim), jnp.float32),
            pltpu.VMEM((chunk, dim), jnp.float32),
            pltpu.VMEM((chunk, acc_w), jnp.float32),
            pltpu.VMEM((heads, LANES), jnp.float32),
            pltpu.VMEM_SHARED((n, acc_w), jnp.float32),
            pltpu.SemaphoreType.DMA,
            pltpu.SemaphoreType.DMA,
            pltpu.SemaphoreType.DMA,
            pltpu.SemaphoreType.DMA,
        ],
    )
    def k(xl_hbm, xr_hbm, src_hbm, dst_hbm, att_hbm, acc_out,
          src0, src1, dst0, dst1, xl0, xl1, xr0, xr1, msg_b, att_v,
          acc_sh, sema0, sema1, semb0, semb1):
        c = lax.axis_index("c")
        s = lax.axis_index("s")
        wid = c * NS + s
        src_i = (src0, src1)
        dst_i = (dst0, dst1)
        xl_b = (xl0, xl1)
        xr_b = (xr0, xr1)
        sema = (sema0, sema1)
        semb = (semb0, semb1)

        pltpu.sync_copy(att_hbm, att_v)

        zero = jnp.zeros((LANES,), jnp.float32)

        def zrow(i, _):
            for j in range(acc_w // LANES):
                msg_b[i, pl.ds(j * LANES, LANES)] = zero
            return 0

        lax.fori_loop(0, chunk, zrow, 0, unroll=False)

        nf, zrem = divmod(rows, chunk)

        def zcopy(i, _):
            pltpu.sync_copy(msg_b, acc_sh.at[pl.ds(s * rows + i * chunk,
                                                   chunk)])
            return 0

        lax.fori_loop(0, nf, zcopy, 0, unroll=False)
        if zrem:
            pltpu.sync_copy(msg_b.at[pl.ds(0, zrem)],
                            acc_sh.at[pl.ds(s * rows + nf * chunk, zrem)])
        plsc.subcore_barrier()

        iota = lax.iota(jnp.int32, LANES)
        base_w = wid * ew
        last = nchunks - 1

        def load_idx(ci, b):
            off = base_w + ci * chunk
            pltpu.sync_copy(src_hbm.at[pl.ds(off, chunk)], src_i[b])
            pltpu.sync_copy(dst_hbm.at[pl.ds(off, chunk)], dst_i[b])

        def start_gather(b):
            pltpu.async_copy(xl_hbm.at[src_i[b]], xl_b[b], sema[b])
            pltpu.async_copy(xr_hbm.at[dst_i[b]], xr_b[b], semb[b])

        def wait_gather(b):
            pltpu.make_async_copy(xl_hbm.at[src_i[b]], xl_b[b],
                                  sema[b]).wait()
            pltpu.make_async_copy(xr_hbm.at[dst_i[b]], xr_b[b],
                                  semb[b]).wait()

        def compute(b):
            xlb = xl_b[b]
            xrb = xr_b[b]
            if single_head:
                @plsc.parallel_loop(0, chunk, step=1, unroll=4)
                def edge_body(ei):
                    u = jnp.zeros((LANES,), jnp.float32)
                    xlv = []
                    for h in range(heads):
                        sl = pl.ds(h * LANES, LANES)
                        xlv.append(xlb[ei, sl])
                        t = xlv[h] + xrb[ei, sl]
                        t = jnp.maximum(t, NEG * t)
                        u = u + t * att_v[h, :]
                    p = jnp.exp(_vsum_bcast(u, iota))
                    for h in range(heads):
                        msg_b[ei, pl.ds(h * LANES, LANES)] = xlv[h] * p
                    msg_b[ei, pl.ds(dim, LANES)] = jnp.where(iota == 0, p,
                                                             0.0)
            else:
                @plsc.parallel_loop(0, chunk, step=1, unroll=4)
                def edge_body(ei):
                    den_v = jnp.zeros((LANES,), jnp.float32)
                    for h in range(heads):
                        sl = pl.ds(h * LANES, LANES)
                        xlv = xlb[ei, sl]
                        t = xlv + xrb[ei, sl]
                        t = jnp.maximum(t, NEG * t)
                        p = jnp.exp(_vsum_bcast(t * att_v[h, :], iota))
                        msg_b[ei, sl] = xlv * p
                        den_v = jnp.where(iota == h, p, den_v)
                    msg_b[ei, pl.ds(dim, LANES)] = den_v

        load_idx(0, 0)
        start_gather(0)

        def pair_body(cb, _):
            for b in range(2):
                ci = cb * 2 + b
                wait_gather(b)
                load_idx(jnp.minimum(ci + 1, last), 1 - b)
                start_gather(1 - b)
                compute(b)
                pltpu.sync_copy(msg_b, acc_sh.at[dst_i[b]], add=True)
            return 0

        lax.fori_loop(0, nchunks // 2, pair_body, 0, unroll=False)
        wait_gather(0)

        plsc.subcore_barrier()
        rw = (n // NS) & ~7
        rem = n - NS * rw
        off = s * rw
        pltpu.sync_copy(acc_sh.at[pl.ds(off, rw)],
                        acc_out.at[c, pl.ds(off, rw)])

        @pl.when(s == NS - 1)
        def _():
            pltpu.sync_copy(acc_sh.at[pl.ds(NS * rw, rem)],
                            acc_out.at[c, pl.ds(NS * rw, rem)])

    return k(xl, xr, src, dst, att)


def kernel(x, edge_index, Wl1, Wr1, att1, bias1, bn_w, bn_b, Wl2, Wr2, att2,
           bias2):
    n = x.shape[0]
    heads, hid = att1.shape

    A1 = (jnp.eye(heads, dtype=jnp.float32)[:, None, :]
          * att1[:, :, None]).reshape(heads * hid, heads)
    E8 = jnp.repeat(jnp.eye(heads, dtype=jnp.float32), hid, axis=1)
    bn_scale = bn_w / jnp.sqrt(1.0 + BN_EPS)
    scale = bn_scale.reshape(1, -1)
    shift = (bias1 * bn_scale + bn_b).reshape(1, -1)
    att2_sc = att2.reshape(-1, LANES)
    att2_tc = att2.reshape(1, -1)
    b2 = bias2.reshape(1, -1)

    src = edge_index[0]
    dst = edge_index[1]

    xl1, xr1 = _tc_lin(x, Wl1, Wr1)
    acc1 = _sc_edges(xl1, xr1, src, dst, att1, heads, 40, False)
    dim1 = heads * hid
    xl2, xr2 = _tc_combine1(
        xl1, xr1, acc1[0, :, :dim1], acc1[1, :, :dim1],
        acc1[0, :, dim1:dim1 + heads], acc1[1, :, dim1:dim1 + heads],
        A1, E8, scale, shift, Wl2, Wr2)

    acc2 = _sc_edges(xl2, xr2, src, dst, att2_sc, 4, 200, True)
    return _tc_final(xl2, xr2, acc2[0, :, :64], acc2[1, :, :64],
                     acc2[0, :, 64:65], acc2[1, :, 64:65],
                     att2_tc, b2)


# speedup vs baseline: 1.3875x; 1.3875x over previous
import functools

import jax
import jax.numpy as jnp
from jax import lax
from jax.experimental import pallas as pl
from jax.experimental.pallas import tpu as pltpu
from jax.experimental.pallas import tpu_sc as plsc

NEG = 0.2
BN_EPS = 1e-5
NC = 2
NS = 16
LANES = 16

_HI = lax.Precision.HIGHEST


def _tc_lin(x, Wl, Wr):
    n, din = x.shape
    dout = Wl.shape[1]
    half = dout // 2
    bn = 1000
    grid = (n // bn,)

    def body(x_ref, wl_ref, wr_ref, xla_ref, xlb_ref, xra_ref, xrb_ref):
        xb = x_ref[...]
        xl = lax.dot(xb, wl_ref[...], precision=_HI,
                     preferred_element_type=jnp.float32)
        xr = lax.dot(xb, wr_ref[...], precision=_HI,
                     preferred_element_type=jnp.float32)
        xla_ref[...] = xl[:, :half]
        xlb_ref[...] = xl[:, half:]
        xra_ref[...] = xr[:, :half]
        xrb_ref[...] = xr[:, half:]

    hspec = pl.BlockSpec((bn, half), lambda i: (i, 0))
    return pl.pallas_call(
        body,
        grid=grid,
        in_specs=[
            pl.BlockSpec((bn, din), lambda i: (i, 0)),
            pl.BlockSpec((din, dout), lambda i: (0, 0)),
            pl.BlockSpec((din, dout), lambda i: (0, 0)),
        ],
        out_specs=[hspec, hspec, hspec, hspec],
        out_shape=[jax.ShapeDtypeStruct((n, half), jnp.float32)] * 4,
    )(x, Wl, Wr)


def _tc_combine1(xla, xlb, xra, xrb, acc_a0, acc_a1, acc_b0, acc_b1, A1, E8,
                 scale, shift, Wl2, Wr2):
    n = xla.shape[0]
    bn = 1000
    grid = (n // bn,)

    def body(xla_ref, xlb_ref, xra_ref, xrb_ref, a0_ref, a1_ref_, b0_ref,
             b1_ref, a1_ref, e8_ref, sc_ref, sh_ref, wl2_ref, wr2_ref,
             xl2_ref, xr2_ref):
        xl = jnp.concatenate([xla_ref[...], xlb_ref[...]], axis=1)
        t = xl + jnp.concatenate([xra_ref[...], xrb_ref[...]], axis=1)
        t = jnp.maximum(t, NEG * t)
        p = jnp.exp(lax.dot(t, a1_ref[...], precision=_HI,
                            preferred_element_type=jnp.float32))
        acca = a0_ref[...] + a1_ref_[...]
        accb = b0_ref[...] + b1_ref[...]
        den8 = jnp.concatenate([acca[:, 64:68], accb[:, 64:68]],
                               axis=1) + p
        e8 = e8_ref[...]
        num = jnp.concatenate([acca[:, :64], accb[:, :64]],
                              axis=1) + lax.dot(
            p, e8, precision=_HI, preferred_element_type=jnp.float32) * xl
        den = lax.dot(den8, e8, precision=_HI,
                      preferred_element_type=jnp.float32)
        o = num / (den + 1e-16)
        o = o * sc_ref[...] + sh_ref[...]
        h = jnp.where(o > 0, o, jnp.exp(o) - 1.0)
        xl2_ref[...] = lax.dot(h, wl2_ref[...], precision=_HI,
                               preferred_element_type=jnp.float32)
        xr2_ref[...] = lax.dot(h, wr2_ref[...], precision=_HI,
                               preferred_element_type=jnp.float32)

    full = lambda shape: pl.BlockSpec(shape, lambda i: tuple(0 for _ in shape))
    return pl.pallas_call(
        body,
        grid=grid,
        in_specs=[
            pl.BlockSpec((bn, 64), lambda i: (i, 0)),
            pl.BlockSpec((bn, 64), lambda i: (i, 0)),
            pl.BlockSpec((bn, 64), lambda i: (i, 0)),
            pl.BlockSpec((bn, 64), lambda i: (i, 0)),
            pl.BlockSpec((bn, 80), lambda i: (i, 0)),
            pl.BlockSpec((bn, 80), lambda i: (i, 0)),
            pl.BlockSpec((bn, 80), lambda i: (i, 0)),
            pl.BlockSpec((bn, 80), lambda i: (i, 0)),
            full((128, 8)),
            full((8, 128)),
            full((1, 128)),
            full((1, 128)),
            full((128, 64)),
            full((128, 64)),
        ],
        out_specs=[
            pl.BlockSpec((bn, 64), lambda i: (i, 0)),
            pl.BlockSpec((bn, 64), lambda i: (i, 0)),
        ],
        out_shape=[
            jax.ShapeDtypeStruct((n, 64), jnp.float32),
            jax.ShapeDtypeStruct((n, 64), jnp.float32),
        ],
    )(xla, xlb, xra, xrb, acc_a0, acc_a1, acc_b0, acc_b1, A1, E8, scale,
      shift, Wl2, Wr2)


def _tc_final(xl2, xr2, num0, num1, den0, den1, att2, bias2):
    n = xl2.shape[0]
    bn = 1000
    grid = (n // bn,)

    def body(xl_ref, xr_ref, n0_ref, n1_ref, d0_ref, d1_ref, a_ref, b_ref,
             out_ref):
        xl = xl_ref[...]
        t = xl + xr_ref[...]
        t = jnp.maximum(t, NEG * t)
        a = jnp.sum(t * a_ref[...], axis=1, keepdims=True)
        p = jnp.exp(a)
        den = d0_ref[...] + d1_ref[...] + p
        num = n0_ref[...] + n1_ref[...] + p * xl
        o = num / (den + 1e-16) + b_ref[...]
        m = jnp.max(o, axis=1, keepdims=True)
        lse = m + jnp.log(jnp.sum(jnp.exp(o - m), axis=1, keepdims=True))
        out_ref[...] = o - lse

    full = lambda shape: pl.BlockSpec(shape, lambda i: tuple(0 for _ in shape))
    return pl.pallas_call(
        body,
        grid=grid,
        in_specs=[
            pl.BlockSpec((bn, 64), lambda i: (i, 0)),
            pl.BlockSpec((bn, 64), lambda i: (i, 0)),
            pl.BlockSpec((bn, 64), lambda i: (i, 0)),
            pl.BlockSpec((bn, 64), lambda i: (i, 0)),
            pl.BlockSpec((bn, 1), lambda i: (i, 0)),
            pl.BlockSpec((bn, 1), lambda i: (i, 0)),
            full((1, 64)),
            full((1, 64)),
        ],
        out_specs=pl.BlockSpec((bn, 64), lambda i: (i, 0)),
        out_shape=jax.ShapeDtypeStruct((n, 64), jnp.float32),
    )(xl2, xr2, num0, num1, den0, den1, att2, bias2)


_GDN = lax.GatherDimensionNumbers(
    offset_dims=(), collapsed_slice_dims=(0,), start_index_map=(0,))


def _lane_shuffle(u, idx):
    return lax.gather(u, idx[:, None], _GDN, (1,),
                      mode=lax.GatherScatterMode.PROMISE_IN_BOUNDS)


def _vsum_bcast(u, iota):
    for k in (1, 2, 4, 8):
        u = u + _lane_shuffle(u, iota ^ k)
    return u

def _sc_edges(xl, xr, src, dst, att, heads, chunk, single_head):
    n = xl.shape[0]
    dim = heads * LANES
    acc_w = dim + LANES
    e = src.shape[0]
    ew = e // (NC * NS)
    nchunks = ew // chunk
    assert nchunks % 2 == 0
    rows = n // NS

    mesh = plsc.VectorSubcoreMesh(core_axis_name="c", subcore_axis_name="s")

    @functools.partial(
        pl.kernel,
        out_type=jax.ShapeDtypeStruct((NC, n, acc_w), jnp.float32),
        mesh=mesh,
        compiler_params=pltpu.CompilerParams(use_tc_tiling_on_sc=False),
        scratch_types=[
            pltpu.VMEM((chunk,), jnp.int32),
            pltpu.VMEM((chunk,), jnp.int32),
            pltpu.VMEM((chunk,), jnp.int32),
            pltpu.VMEM((chunk,), jnp.int32),
            pltpu.VMEM((chunk, dim), jnp.float32),
            pltpu.VMEM((chunk, dim), jnp.float32),
            pltpu.VMEM((chunk, dim), jnp.float32),
            pltpu.VMEM((chunk, dim), jnp.float32),
            pltpu.VMEM((chunk, acc_w), jnp.float32),
            pltpu.VMEM((heads, LANES), jnp.float32),
            pltpu.VMEM_SHARED((n, acc_w), jnp.float32),
            pltpu.SemaphoreType.DMA,
            pltpu.SemaphoreType.DMA,
            pltpu.SemaphoreType.DMA,
            pltpu.SemaphoreType.DMA,
        ],
    )
    def k(xl_hbm, xr_hbm, src_hbm, dst_hbm, att_hbm, acc_out,
          src0, src1, dst0, dst1, xl0, xl1, xr0, xr1, msg_b, att_v,
          acc_sh, sema0, sema1, semb0, semb1):
        c = lax.axis_index("c")
        s = lax.axis_index("s")
        wid = c * NS + s
        src_i = (src0, src1)
        dst_i = (dst0, dst1)
        xl_b = (xl0, xl1)
        xr_b = (xr0, xr1)
        sema = (sema0, sema1)
        semb = (semb0, semb1)

        pltpu.sync_copy(att_hbm, att_v)

        zero = jnp.zeros((LANES,), jnp.float32)

        def zrow(i, _):
            for j in range(acc_w // LANES):
                msg_b[i, pl.ds(j * LANES, LANES)] = zero
            return 0

        lax.fori_loop(0, chunk, zrow, 0, unroll=False)

        nf, zrem = divmod(rows, chunk)

        def zcopy(i, _):
            pltpu.sync_copy(msg_b, acc_sh.at[pl.ds(s * rows + i * chunk,
                                                   chunk)])
            return 0

        lax.fori_loop(0, nf, zcopy, 0, unroll=False)
        if zrem:
            pltpu.sync_copy(msg_b.at[pl.ds(0, zrem)],
                            acc_sh.at[pl.ds(s * rows + nf * chunk, zrem)])
        plsc.subcore_barrier()

        iota = lax.iota(jnp.int32, LANES)
        base_w = wid * ew
        last = nchunks - 1

        def load_idx(ci, b):
            off = base_w + ci * chunk
            pltpu.sync_copy(src_hbm.at[pl.ds(off, chunk)], src_i[b])
            pltpu.sync_copy(dst_hbm.at[pl.ds(off, chunk)], dst_i[b])

        def start_gather(b):
            pltpu.async_copy(xl_hbm.at[src_i[b]], xl_b[b], sema[b])
            pltpu.async_copy(xr_hbm.at[dst_i[b]], xr_b[b], semb[b])

        def wait_gather(b):
            pltpu.make_async_copy(xl_hbm.at[src_i[b]], xl_b[b],
                                  sema[b]).wait()
            pltpu.make_async_copy(xr_hbm.at[dst_i[b]], xr_b[b],
                                  semb[b]).wait()

        def compute(b):
            xlb = xl_b[b]
            xrb = xr_b[b]
            if single_head:
                @plsc.parallel_loop(0, chunk, step=1, unroll=4)
                def edge_body(ei):
                    u = jnp.zeros((LANES,), jnp.float32)
                    xlv = []
                    for h in range(heads):
                        sl = pl.ds(h * LANES, LANES)
                        xlv.append(xlb[ei, sl])
                        t = xlv[h] + xrb[ei, sl]
                        t = jnp.maximum(t, NEG * t)
                        u = u + t * att_v[h, :]
                    p = jnp.exp(_vsum_bcast(u, iota))
                    for h in range(heads):
                        msg_b[ei, pl.ds(h * LANES, LANES)] = xlv[h] * p
                    msg_b[ei, pl.ds(dim, LANES)] = jnp.where(iota == 0, p,
                                                             0.0)
            else:
                @plsc.parallel_loop(0, chunk, step=1, unroll=4)
                def edge_body(ei):
                    den_v = jnp.zeros((LANES,), jnp.float32)
                    for h in range(heads):
                        sl = pl.ds(h * LANES, LANES)
                        xlv = xlb[ei, sl]
                        t = xlv + xrb[ei, sl]
                        t = jnp.maximum(t, NEG * t)
                        p = jnp.exp(_vsum_bcast(t * att_v[h, :], iota))
                        msg_b[ei, sl] = xlv * p
                        den_v = jnp.where(iota == h, p, den_v)
                    msg_b[ei, pl.ds(dim, LANES)] = den_v

        load_idx(0, 0)
        start_gather(0)

        def pair_body(cb, _):
            for b in range(2):
                ci = cb * 2 + b
                wait_gather(b)
                load_idx(jnp.minimum(ci + 1, last), 1 - b)
                start_gather(1 - b)
                compute(b)
                pltpu.sync_copy(msg_b, acc_sh.at[dst_i[b]], add=True)
            return 0

        lax.fori_loop(0, nchunks // 2, pair_body, 0, unroll=False)
        wait_gather(0)

        plsc.subcore_barrier()
        rw = (n // NS) & ~7
        rem = n - NS * rw
        off = s * rw
        pltpu.sync_copy(acc_sh.at[pl.ds(off, rw)],
                        acc_out.at[c, pl.ds(off, rw)])

        @pl.when(s == NS - 1)
        def _():
            pltpu.sync_copy(acc_sh.at[pl.ds(NS * rw, rem)],
                            acc_out.at[c, pl.ds(NS * rw, rem)])

    return k(xl, xr, src, dst, att)


def kernel(x, edge_index, Wl1, Wr1, att1, bias1, bn_w, bn_b, Wl2, Wr2, att2,
           bias2):
    n = x.shape[0]
    heads, hid = att1.shape

    A1 = (jnp.eye(heads, dtype=jnp.float32)[:, None, :]
          * att1[:, :, None]).reshape(heads * hid, heads)
    E8 = jnp.repeat(jnp.eye(heads, dtype=jnp.float32), hid, axis=1)
    bn_scale = bn_w / jnp.sqrt(1.0 + BN_EPS)
    scale = bn_scale.reshape(1, -1)
    shift = (bias1 * bn_scale + bn_b).reshape(1, -1)
    att2_sc = att2.reshape(-1, LANES)
    att2_tc = att2.reshape(1, -1)
    b2 = bias2.reshape(1, -1)

    src = edge_index[0]
    dst = edge_index[1]

    xla, xlb, xra, xrb = _tc_lin(x, Wl1, Wr1)
    hh = heads // 2
    acc_a = _sc_edges(xla, xra, src, dst, att1[:hh], hh, 200, False)
    acc_b = _sc_edges(xlb, xrb, src, dst, att1[hh:], hh, 200, False)
    xl2, xr2 = _tc_combine1(
        xla, xlb, xra, xrb,
        acc_a[0], acc_a[1], acc_b[0], acc_b[1],
        A1, E8, scale, shift, Wl2, Wr2)

    acc2 = _sc_edges(xl2, xr2, src, dst, att2_sc, 4, 200, True)
    return _tc_final(xl2, xr2, acc2[0, :, :64], acc2[1, :, :64],
                     acc2[0, :, 64:65], acc2[1, :, 64:65],
                     att2_tc, b2)


# speedup vs baseline: 1.7942x; 1.2930x over previous
import functools

import jax
import jax.numpy as jnp
from jax import lax
from jax.experimental import pallas as pl
from jax.experimental.pallas import tpu as pltpu
from jax.experimental.pallas import tpu_sc as plsc

NEG = 0.2
BN_EPS = 1e-5
NC = 2
NS = 16
LANES = 16

_HI = lax.Precision.HIGHEST


def _tc_lin(x, Wl, Wr):
    n, din = x.shape
    dout = Wl.shape[1]
    half = dout // 2
    bn = 1000
    grid = (n // bn,)

    def body(x_ref, wl_ref, wr_ref, xla_ref, xlb_ref, xra_ref, xrb_ref):
        xb = x_ref[...]
        xl = lax.dot(xb, wl_ref[...], precision=_HI,
                     preferred_element_type=jnp.float32)
        xr = lax.dot(xb, wr_ref[...], precision=_HI,
                     preferred_element_type=jnp.float32)
        xla_ref[...] = xl[:, :half]
        xlb_ref[...] = xl[:, half:]
        xra_ref[...] = xr[:, :half]
        xrb_ref[...] = xr[:, half:]

    hspec = pl.BlockSpec((bn, half), lambda i: (i, 0))
    return pl.pallas_call(
        body,
        grid=grid,
        in_specs=[
            pl.BlockSpec((bn, din), lambda i: (i, 0)),
            pl.BlockSpec((din, dout), lambda i: (0, 0)),
            pl.BlockSpec((din, dout), lambda i: (0, 0)),
        ],
        out_specs=[hspec, hspec, hspec, hspec],
        out_shape=[jax.ShapeDtypeStruct((n, half), jnp.float32)] * 4,
    )(x, Wl, Wr)


def _tc_combine1(xla, xlb, xra, xrb, acc_a0, acc_a1, acc_b0, acc_b1, A1, E8,
                 scale, shift, Wl2, Wr2):
    n = xla.shape[0]
    bn = 1000
    grid = (n // bn,)

    def body(xla_ref, xlb_ref, xra_ref, xrb_ref, a0_ref, a1_ref_, b0_ref,
             b1_ref, a1_ref, e8_ref, sc_ref, sh_ref, wl2_ref, wr2_ref,
             xl2_ref, xr2_ref):
        xl = jnp.concatenate([xla_ref[...], xlb_ref[...]], axis=1)
        t = xl + jnp.concatenate([xra_ref[...], xrb_ref[...]], axis=1)
        t = jnp.maximum(t, NEG * t)
        p = jnp.exp(lax.dot(t, a1_ref[...], precision=_HI,
                            preferred_element_type=jnp.float32))
        acca = a0_ref[...] + a1_ref_[...]
        accb = b0_ref[...] + b1_ref[...]
        den8 = jnp.concatenate([acca[:, 64:68], accb[:, 64:68]],
                               axis=1) + p
        e8 = e8_ref[...]
        num = jnp.concatenate([acca[:, :64], accb[:, :64]],
                              axis=1) + lax.dot(
            p, e8, precision=_HI, preferred_element_type=jnp.float32) * xl
        den = lax.dot(den8, e8, precision=_HI,
                      preferred_element_type=jnp.float32)
        o = num / (den + 1e-16)
        o = o * sc_ref[...] + sh_ref[...]
        h = jnp.where(o > 0, o, jnp.exp(o) - 1.0)
        xl2_ref[...] = lax.dot(h, wl2_ref[...], precision=_HI,
                               preferred_element_type=jnp.float32)
        xr2_ref[...] = lax.dot(h, wr2_ref[...], precision=_HI,
                               preferred_element_type=jnp.float32)

    full = lambda shape: pl.BlockSpec(shape, lambda i: tuple(0 for _ in shape))
    return pl.pallas_call(
        body,
        grid=grid,
        in_specs=[
            pl.BlockSpec((bn, 64), lambda i: (i, 0)),
            pl.BlockSpec((bn, 64), lambda i: (i, 0)),
            pl.BlockSpec((bn, 64), lambda i: (i, 0)),
            pl.BlockSpec((bn, 64), lambda i: (i, 0)),
            pl.BlockSpec((bn, 80), lambda i: (i, 0)),
            pl.BlockSpec((bn, 80), lambda i: (i, 0)),
            pl.BlockSpec((bn, 80), lambda i: (i, 0)),
            pl.BlockSpec((bn, 80), lambda i: (i, 0)),
            full((128, 8)),
            full((8, 128)),
            full((1, 128)),
            full((1, 128)),
            full((128, 64)),
            full((128, 64)),
        ],
        out_specs=[
            pl.BlockSpec((bn, 64), lambda i: (i, 0)),
            pl.BlockSpec((bn, 64), lambda i: (i, 0)),
        ],
        out_shape=[
            jax.ShapeDtypeStruct((n, 64), jnp.float32),
            jax.ShapeDtypeStruct((n, 64), jnp.float32),
        ],
    )(xla, xlb, xra, xrb, acc_a0, acc_a1, acc_b0, acc_b1, A1, E8, scale,
      shift, Wl2, Wr2)


def _tc_final(xl2, xr2, num0, num1, den0, den1, att2, bias2):
    n = xl2.shape[0]
    bn = 1000
    grid = (n // bn,)

    def body(xl_ref, xr_ref, n0_ref, n1_ref, d0_ref, d1_ref, a_ref, b_ref,
             out_ref):
        xl = xl_ref[...]
        t = xl + xr_ref[...]
        t = jnp.maximum(t, NEG * t)
        a = jnp.sum(t * a_ref[...], axis=1, keepdims=True)
        p = jnp.exp(a)
        den = d0_ref[...] + d1_ref[...] + p
        num = n0_ref[...] + n1_ref[...] + p * xl
        o = num / (den + 1e-16) + b_ref[...]
        m = jnp.max(o, axis=1, keepdims=True)
        lse = m + jnp.log(jnp.sum(jnp.exp(o - m), axis=1, keepdims=True))
        out_ref[...] = o - lse

    full = lambda shape: pl.BlockSpec(shape, lambda i: tuple(0 for _ in shape))
    return pl.pallas_call(
        body,
        grid=grid,
        in_specs=[
            pl.BlockSpec((bn, 64), lambda i: (i, 0)),
            pl.BlockSpec((bn, 64), lambda i: (i, 0)),
            pl.BlockSpec((bn, 64), lambda i: (i, 0)),
            pl.BlockSpec((bn, 64), lambda i: (i, 0)),
            pl.BlockSpec((bn, 1), lambda i: (i, 0)),
            pl.BlockSpec((bn, 1), lambda i: (i, 0)),
            full((1, 64)),
            full((1, 64)),
        ],
        out_specs=pl.BlockSpec((bn, 64), lambda i: (i, 0)),
        out_shape=jax.ShapeDtypeStruct((n, 64), jnp.float32),
    )(xl2, xr2, num0, num1, den0, den1, att2, bias2)


_GDN = lax.GatherDimensionNumbers(
    offset_dims=(), collapsed_slice_dims=(0,), start_index_map=(0,))


def _lane_shuffle(u, idx):
    return lax.gather(u, idx[:, None], _GDN, (1,),
                      mode=lax.GatherScatterMode.PROMISE_IN_BOUNDS)


def _vsum_bcast(u, iota):
    for k in (1, 2, 4, 8):
        u = u + _lane_shuffle(u, iota ^ k)
    return u

def _sc_edges(xl, xr, src, dst, att, heads, chunk, single_head):
    n = xl.shape[0]
    dim = heads * LANES
    acc_w = dim + LANES
    e = src.shape[0]
    ew = e // (NC * NS)
    nchunks = ew // chunk
    rows = n // NS

    mesh = plsc.VectorSubcoreMesh(core_axis_name="c", subcore_axis_name="s")

    @functools.partial(
        pl.kernel,
        out_type=jax.ShapeDtypeStruct((NC, n, acc_w), jnp.float32),
        mesh=mesh,
        compiler_params=pltpu.CompilerParams(use_tc_tiling_on_sc=False),
        scratch_types=[
            pltpu.VMEM((ew,), jnp.int32),
            pltpu.VMEM((ew,), jnp.int32),
            pltpu.VMEM((chunk, dim), jnp.float32),
            pltpu.VMEM((chunk, dim), jnp.float32),
            pltpu.VMEM((chunk, dim), jnp.float32),
            pltpu.VMEM((chunk, dim), jnp.float32),
            pltpu.VMEM((chunk, acc_w), jnp.float32),
            pltpu.VMEM((chunk, acc_w), jnp.float32),
            pltpu.VMEM((heads, LANES), jnp.float32),
            pltpu.VMEM_SHARED((n, acc_w), jnp.float32),
            pltpu.SemaphoreType.DMA,
            pltpu.SemaphoreType.DMA,
            pltpu.SemaphoreType.DMA,
            pltpu.SemaphoreType.DMA,
            pltpu.SemaphoreType.DMA,
            pltpu.SemaphoreType.DMA,
        ],
    )
    def k(xl_hbm, xr_hbm, src_hbm, dst_hbm, att_hbm, acc_out,
          src_i, dst_i, xl0, xl1, xr0, xr1, msg0, msg1, att_v,
          acc_sh, sema0, sema1, semb0, semb1, sems0, sems1):
        c = lax.axis_index("c")
        s = lax.axis_index("s")
        wid = c * NS + s
        xl_b = (xl0, xl1)
        xr_b = (xr0, xr1)
        msg_b = (msg0, msg1)
        sema = (sema0, sema1)
        semb = (semb0, semb1)
        sems = (sems0, sems1)

        pltpu.sync_copy(att_hbm, att_v)
        base_w = wid * ew
        pltpu.sync_copy(src_hbm.at[pl.ds(base_w, ew)], src_i)
        pltpu.sync_copy(dst_hbm.at[pl.ds(base_w, ew)], dst_i)

        zero = jnp.zeros((LANES,), jnp.float32)

        def zrow(i, _):
            for j in range(acc_w // LANES):
                msg0[i, pl.ds(j * LANES, LANES)] = zero
                msg1[i, pl.ds(j * LANES, LANES)] = zero
            return 0

        lax.fori_loop(0, chunk, zrow, 0, unroll=False)

        nf, zrem = divmod(rows, chunk)

        def zcopy(i, _):
            pltpu.sync_copy(msg0, acc_sh.at[pl.ds(s * rows + i * chunk,
                                                  chunk)])
            return 0

        lax.fori_loop(0, nf, zcopy, 0, unroll=False)
        if zrem:
            pltpu.sync_copy(msg0.at[pl.ds(0, zrem)],
                            acc_sh.at[pl.ds(s * rows + nf * chunk, zrem)])
        plsc.subcore_barrier()

        iota = lax.iota(jnp.int32, LANES)

        def sidx(ci):
            return src_i.at[pl.ds(ci * chunk, chunk)]

        def didx(ci):
            return dst_i.at[pl.ds(ci * chunk, chunk)]

        def start_gather(ci, b):
            pltpu.async_copy(xl_hbm.at[sidx(ci)], xl_b[b], sema[b])
            pltpu.async_copy(xr_hbm.at[didx(ci)], xr_b[b], semb[b])

        def wait_gather(ci, b):
            pltpu.make_async_copy(xl_hbm.at[sidx(ci)], xl_b[b],
                                  sema[b]).wait()
            pltpu.make_async_copy(xr_hbm.at[didx(ci)], xr_b[b],
                                  semb[b]).wait()

        def start_scatter(ci, b):
            pltpu.async_copy(msg_b[b], acc_sh.at[didx(ci)], sems[b],
                             add=True)

        def wait_scatter(ci, b):
            pltpu.make_async_copy(msg_b[b], acc_sh.at[didx(ci)],
                                  sems[b]).wait()

        def compute(b):
            xlb = xl_b[b]
            xrb = xr_b[b]
            msgb = msg_b[b]
            if single_head:
                @plsc.parallel_loop(0, chunk, step=1, unroll=4)
                def edge_body(ei):
                    u = jnp.zeros((LANES,), jnp.float32)
                    xlv = []
                    for h in range(heads):
                        sl = pl.ds(h * LANES, LANES)
                        xlv.append(xlb[ei, sl])
                        t = xlv[h] + xrb[ei, sl]
                        t = jnp.maximum(t, NEG * t)
                        u = u + t * att_v[h, :]
                    p = jnp.exp(_vsum_bcast(u, iota))
                    for h in range(heads):
                        msgb[ei, pl.ds(h * LANES, LANES)] = xlv[h] * p
                    msgb[ei, pl.ds(dim, LANES)] = jnp.where(iota == 0, p,
                                                            0.0)
            else:
                @plsc.parallel_loop(0, chunk, step=1, unroll=4)
                def edge_body(ei):
                    den_v = jnp.zeros((LANES,), jnp.float32)
                    for h in range(heads):
                        sl = pl.ds(h * LANES, LANES)
                        xlv = xlb[ei, sl]
                        t = xlv + xrb[ei, sl]
                        t = jnp.maximum(t, NEG * t)
                        p = jnp.exp(_vsum_bcast(t * att_v[h, :], iota))
                        msgb[ei, sl] = xlv * p
                        den_v = jnp.where(iota == h, p, den_v)
                    msgb[ei, pl.ds(dim, LANES)] = den_v

        start_scatter(0, 0)
        start_scatter(0, 1)
        start_gather(0, 0)

        def pair_body(cb, _):
            for b in range(2):
                ci = cb * 2 + b
                wait_gather(ci, b)
                start_gather(ci + 1, 1 - b)
                wait_scatter(ci, b)
                compute(b)
                start_scatter(ci, b)
            return 0

        m = (nchunks - 1) // 2
        lax.fori_loop(0, m, pair_body, 0, unroll=False)
        for ci in range(2 * m, nchunks):
            b = ci % 2
            wait_gather(ci, b)
            if ci + 1 < nchunks:
                start_gather(ci + 1, 1 - b)
            wait_scatter(ci, b)
            compute(b)
            start_scatter(ci, b)
        wait_scatter(nchunks - 2, 0 if nchunks % 2 == 0 else 1)
        wait_scatter(nchunks - 1, 1 if nchunks % 2 == 0 else 0)

        plsc.subcore_barrier()
        rw = (n // NS) & ~7
        rem = n - NS * rw
        off = s * rw
        pltpu.sync_copy(acc_sh.at[pl.ds(off, rw)],
                        acc_out.at[c, pl.ds(off, rw)])

        @pl.when(s == NS - 1)
        def _():
            pltpu.sync_copy(acc_sh.at[pl.ds(NS * rw, rem)],
                            acc_out.at[c, pl.ds(NS * rw, rem)])

    return k(xl, xr, src, dst, att)


def kernel(x, edge_index, Wl1, Wr1, att1, bias1, bn_w, bn_b, Wl2, Wr2, att2,
           bias2):
    n = x.shape[0]
    heads, hid = att1.shape

    A1 = (jnp.eye(heads, dtype=jnp.float32)[:, None, :]
          * att1[:, :, None]).reshape(heads * hid, heads)
    E8 = jnp.repeat(jnp.eye(heads, dtype=jnp.float32), hid, axis=1)
    bn_scale = bn_w / jnp.sqrt(1.0 + BN_EPS)
    scale = bn_scale.reshape(1, -1)
    shift = (bias1 * bn_scale + bn_b).reshape(1, -1)
    att2_sc = att2.reshape(-1, LANES)
    att2_tc = att2.reshape(1, -1)
    b2 = bias2.reshape(1, -1)

    src = edge_index[0]
    dst = edge_index[1]

    xla, xlb, xra, xrb = _tc_lin(x, Wl1, Wr1)
    hh = heads // 2
    acc_a = _sc_edges(xla, xra, src, dst, att1[:hh], hh, 80, False)
    acc_b = _sc_edges(xlb, xrb, src, dst, att1[hh:], hh, 80, False)
    xl2, xr2 = _tc_combine1(
        xla, xlb, xra, xrb,
        acc_a[0], acc_a[1], acc_b[0], acc_b[1],
        A1, E8, scale, shift, Wl2, Wr2)

    acc2 = _sc_edges(xl2, xr2, src, dst, att2_sc, 4, 80, True)
    return _tc_final(xl2, xr2, acc2[0, :, :64], acc2[1, :, :64],
                     acc2[0, :, 64:65], acc2[1, :, 64:65],
                     att2_tc, b2)
